# baseline (speedup 1.0000x reference)
import functools

import jax
import jax.numpy as jnp
from jax import lax
from jax.experimental import pallas as pl
from jax.experimental.pallas import tpu as pltpu
from jax.experimental.pallas import tpu_sc as plsc

N_NODES = 10000
E_EDGES = 320000
D_NODE = 128
D_EDGE = 16
H_DIM = 128
HH = H_DIM // 4

N_PAD = 10240
E_PAD = 327680
BE = 4096
BN = 2000
BISECT_ITERS = 24

NSEG_R = N_PAD // 16
BIS_TILE_R = E_PAD // 16 // 16
MSG_CHUNKS = E_PAD // 32 // 128


def _i32(v):
    return jnp.asarray(v, jnp.int32)


def _edge_kernel(ea_ref, We_ref, be_ref, W2_ref, b2_ref, Wa_ref, wa_ref, ca_ref,
                 a_ref, e2a_ref, e2b_ref, e2c_ref, e2d_ref):
    ea = ea_ref[...]
    eo = jnp.maximum(jnp.dot(ea, We_ref[...],
                             preferred_element_type=jnp.float32) + be_ref[...], 0.0)
    e2 = jnp.dot(eo, W2_ref[...],
                 preferred_element_type=jnp.float32) + b2_ref[...]
    e2a_ref[...] = e2[:, :HH]
    e2b_ref[...] = e2[:, HH:2 * HH]
    e2c_ref[...] = e2[:, 2 * HH:3 * HH]
    e2d_ref[...] = e2[:, 3 * HH:]
    v = jnp.sum(Wa_ref[...] * wa_ref[...], axis=1)
    aa = jnp.sum(ea * v[None, :], axis=1) + ca_ref[0]
    a_ref[...] = jnp.where(aa > 0, aa, 0.01 * aa)


def _matmul_bias_kernel(x_ref, W_ref, b_ref, o_ref):
    o_ref[...] = jnp.dot(x_ref[...], W_ref[...],
                         preferred_element_type=jnp.float32) + b_ref[...]


def _final_kernel(x_ref, hn0_ref, hn1_ref, hn2_ref, hn3_ref,
                  hn4_ref, hn5_ref, hn6_ref, hn7_ref,
                  W2q0_ref, W2q1_ref, W2q2_ref, W2q3_ref,
                  W1_ref, b_ref, o_ref):
    acc = jnp.dot(x_ref[...], W1_ref[...], preferred_element_type=jnp.float32)
    acc += jnp.dot(hn0_ref[0, 0] + hn4_ref[0, 0], W2q0_ref[...],
                   preferred_element_type=jnp.float32)
    acc += jnp.dot(hn1_ref[0, 0] + hn5_ref[0, 0], W2q1_ref[...],
                   preferred_element_type=jnp.float32)
    acc += jnp.dot(hn2_ref[0, 0] + hn6_ref[0, 0], W2q2_ref[...],
                   preferred_element_type=jnp.float32)
    acc += jnp.dot(hn3_ref[0, 0] + hn7_ref[0, 0], W2q3_ref[...],
                   preferred_element_type=jnp.float32)
    o_ref[...] = jnp.maximum(acc + b_ref[...], 0.0)


def _bisect_kernel(a16, d16, tau_hbm,
                   a_loc, d_loc, mid_loc, s_loc, lo_loc, hi_loc, zero_loc,
                   idx_loc, tau_loc, mm_loc, mmall_loc, s_sh, mm_sh):
    c = lax.axis_index("c")
    w = lax.axis_index("s")
    iota = jnp.arange(16, dtype=jnp.int32)

    pltpu.sync_copy(a16.at[pl.ds(w * BIS_TILE_R, BIS_TILE_R)], a_loc)
    pltpu.sync_copy(d16.at[pl.ds(w * BIS_TILE_R, BIS_TILE_R)], d_loc)

    def init_body(g, _):
        zero_loc[g] = jnp.zeros((16,), jnp.float32)
        return 0
    lax.fori_loop(0, NSEG_R, init_body, 0)

    def idx_body(g, vec):
        idx_loc[pl.ds(g * 16, 16)] = vec
        return vec + 16
    lax.fori_loop(0, NSEG_R // 16, idx_body, iota)

    def mm_body(g, carry):
        mn, mx = carry
        av = a_loc[g]
        return jnp.minimum(mn, av), jnp.maximum(mx, av)
    mn, mx = lax.fori_loop(0, BIS_TILE_R, mm_body,
                           (jnp.full((16,), jnp.inf, jnp.float32),
                            jnp.full((16,), -jnp.inf, jnp.float32)))
    gmn = jnp.min(mn)
    gmx = jnp.max(mx)
    mm_loc[0] = jnp.where(iota == 0, gmn, -gmx)
    pltpu.sync_copy(mm_loc, mm_sh.at[pl.ds(w, 1)])
    plsc.subcore_barrier()
    pltpu.sync_copy(mm_sh, mmall_loc)

    def mm_red(t, acc):
        return jnp.minimum(acc, mmall_loc[t])
    acc = lax.fori_loop(0, 16, mm_red, jnp.full((16,), jnp.inf, jnp.float32))
    inf = jnp.float32(jnp.inf)
    gmin = jnp.min(jnp.where(iota == 0, acc, inf))
    gmax = -jnp.min(jnp.where(iota == 1, acc, inf))

    def lohi_body(g, _):
        lo_loc[g] = jnp.full((16,), gmin - 1.0, jnp.float32)
        hi_loc[g] = jnp.full((16,), gmax, jnp.float32)
        mid_loc[g] = jnp.full((16,), 0.5 * (gmin - 1.0 + gmax), jnp.float32)
        s_loc[g] = jnp.zeros((16,), jnp.float32)
        return 0
    lax.fori_loop(0, NSEG_R, lohi_body, 0)

    def iter_body(_, carry):
        def edge_body(g, _c):
            idxv = d_loc[g]
            av = a_loc[g]
            row = jax.lax.shift_right_logical(idxv, _i32(4))
            lane = jnp.bitwise_and(idxv, _i32(15))
            mv = plsc.load_gather(mid_loc, [row, lane])
            contrib = jnp.maximum(av - mv, 0.0)
            plsc.addupdate_scatter(s_loc, [row, lane], contrib)
            return 0
        lax.fori_loop(0, BIS_TILE_R, edge_body, 0)

        plsc.subcore_barrier()

        @pl.when(w == 0)
        def _zero():
            pltpu.sync_copy(zero_loc, s_sh)
        plsc.subcore_barrier()
        pltpu.sync_copy(s_loc, s_sh.at[idx_loc], add=True)
        plsc.subcore_barrier()
        pltpu.sync_copy(s_sh, s_loc)

        def upd_body(g, _c):
            ge = s_loc[g] >= 1.0
            midv = mid_loc[g]
            lo = jnp.where(ge, midv, lo_loc[g])
            hi = jnp.where(ge, hi_loc[g], midv)
            lo_loc[g] = lo
            hi_loc[g] = hi
            mid_loc[g] = 0.5 * (lo + hi)
            s_loc[g] = jnp.zeros((16,), jnp.float32)
            return 0
        lax.fori_loop(0, NSEG_R, upd_body, 0)
        return 0

    lax.fori_loop(0, BISECT_ITERS, iter_body, 0)

    def tau_body(j, _c):
        g = w * (NSEG_R // 16) + j
        tau_loc[j] = 0.5 * (lo_loc[g] + hi_loc[g])
        return 0
    lax.fori_loop(0, NSEG_R // 16, tau_body, 0)

    @pl.when(c == 0)
    def _write():
        pltpu.sync_copy(tau_loc, tau_hbm.at[pl.ds(w * (NSEG_R // 16),
                                                  NSEG_R // 16)])


def _message_kernel(y1a_hbm, y1b_hbm, y1c_hbm, y1d_hbm,
                    e2a_hbm, e2b_hbm, e2c_hbm, e2d_hbm, a1_hbm, src1_hbm,
                    dst1_hbm, tau_hbm, hn_hbm,
                    tau_loc, a_loc, src_loc, dst_loc, alpha_c, src_c, dst_c,
                    eid_c, dst_chunk, g_buf, e_buf, zero_big, hn_sh,
                    sem, sem2):
    c = lax.axis_index("c")
    s_ = lax.axis_index("s")
    wid = c * 16 + s_
    e_base = wid * (MSG_CHUNKS * 128)
    iota = jnp.arange(16, dtype=jnp.int32)

    pltpu.sync_copy(tau_hbm, tau_loc)
    pltpu.sync_copy(a1_hbm.at[pl.ds(e_base, MSG_CHUNKS * 128)], a_loc)
    pltpu.sync_copy(src1_hbm.at[pl.ds(e_base, MSG_CHUNKS * 128)], src_loc)
    pltpu.sync_copy(dst1_hbm.at[pl.ds(e_base, MSG_CHUNKS * 128)], dst_loc)

    def zb(i, _c):
        def zq(q, _cc):
            zero_big[i, pl.ds(q * 16, 16)] = jnp.zeros((16,), jnp.float32)
            return 0
        lax.fori_loop(0, HH // 16, zq, 0)
        return 0
    lax.fori_loop(0, 128, zb, 0)

    def comp_body(g, carry):
        pos, ev = carry
        dv = dst_loc[pl.ds(g * 16, 16)]
        av = a_loc[pl.ds(g * 16, 16)]
        sv = src_loc[pl.ds(g * 16, 16)]
        row = jax.lax.shift_right_logical(dv, _i32(4))
        lane = jnp.bitwise_and(dv, _i32(15))
        tv = plsc.load_gather(tau_loc, [row, lane])
        alpha = jnp.maximum(av - tv, 0.0)
        mask = alpha > 0.0
        plsc.store_compressed(alpha_c.at[pl.ds(pos, 16)], x=alpha, mask=mask)
        plsc.store_compressed(src_c.at[pl.ds(pos, 16)], x=sv, mask=mask)
        plsc.store_compressed(dst_c.at[pl.ds(pos, 16)], x=dv, mask=mask)
        plsc.store_compressed(eid_c.at[pl.ds(pos, 16)], x=ev, mask=mask)
        npos = pos + jnp.max(plsc.all_reduce_population_count(mask))
        return npos, ev + 16
    cnt, _ = lax.fori_loop(
        0, MSG_CHUNKS * 8, comp_body,
        (_i32(0), jnp.full((16,), e_base, jnp.int32) + iota))

    def pad_body(j, _c):
        at = pl.ds(cnt + j * 16, 16)
        alpha_c[at] = jnp.zeros((16,), jnp.float32)
        src_c[at] = jnp.zeros((16,), jnp.int32)
        dst_c[at] = jnp.full((16,), N_PAD - 1, jnp.int32)
        eid_c[at] = jnp.zeros((16,), jnp.int32)
        return 0
    lax.fori_loop(0, 8, pad_body, 0)

    for ha, (y1h, e2h) in enumerate(((y1a_hbm, e2a_hbm), (y1b_hbm, e2b_hbm),
                                     (y1c_hbm, e2c_hbm), (y1d_hbm, e2d_hbm))):
        def zs(j, _c):
            pltpu.sync_copy(zero_big,
                            hn_sh.at[pl.ds(s_ * 640 + j * 128, 128)])
            return 0
        lax.fori_loop(0, 5, zs, 0)
        plsc.subcore_barrier()

        def chunk_body(ch, _c):
            @pl.when(ch * 128 < cnt)
            def _do():
                def cp(l, _cc):
                    dst_chunk[pl.ds(l * 16, 16)] = (
                        dst_c[pl.ds(ch * 128 + l * 16, 16)])
                    return 0
                lax.fori_loop(0, 8, cp, 0)

                cp1 = pltpu.async_copy(
                    y1h.at[src_c.at[pl.ds(ch * 128, 128)]], g_buf, sem)
                cp2 = pltpu.async_copy(
                    e2h.at[eid_c.at[pl.ds(ch * 128, 128)]], e_buf, sem2)
                cp1.wait()
                cp2.wait()

                def rowb(r, rfull):
                    ar = plsc.load_gather(alpha_c.at[pl.ds(ch * 128, 128)],
                                          [rfull])

                    def qb(q, _ccc):
                        mv = jnp.maximum(g_buf[r, pl.ds(q * 16, 16)]
                                         + e_buf[r, pl.ds(q * 16, 16)],
                                         0.0) * ar
                        g_buf[r, pl.ds(q * 16, 16)] = mv
                        return 0
                    lax.fori_loop(0, HH // 16, qb, 0)
                    return rfull + 1
                lax.fori_loop(0, 128, rowb, jnp.zeros((16,), jnp.int32))

                pltpu.sync_copy(g_buf, hn_sh.at[dst_chunk], add=True)
            return 0
        lax.fori_loop(0, MSG_CHUNKS, chunk_body, 0)

        plsc.subcore_barrier()
        pltpu.sync_copy(hn_sh.at[pl.ds(s_ * 640, 640)],
                        hn_hbm.at[c, ha, pl.ds(s_ * 640, 640)])
        plsc.subcore_barrier()


def kernel(x, edge_index, edge_attr, W_edge, b_edge, W_eattn, b_eattn, w_attn,
           W_eout, b_eout, W_node, b_node):
    edge_index = edge_index.astype(jnp.int32)
    with jax.enable_x64(False):
        return _kernel_impl(x, edge_index, edge_attr, W_edge, b_edge, W_eattn,
                            b_eattn, w_attn, W_eout, b_eout, W_node, b_node)


def _kernel_impl(x, edge_index, edge_attr, W_edge, b_edge, W_eattn, b_eattn,
                 w_attn, W_eout, b_eout, W_node, b_node):
    x = x.astype(jnp.float32)
    src = edge_index[0]
    dst = edge_index[1]
    edge_attr = edge_attr.astype(jnp.float32)

    pad = E_PAD - E_EDGES
    src_p = jnp.concatenate([src, jnp.zeros((pad,), jnp.int32)])
    dst_p = jnp.concatenate([dst, jnp.full((pad,), N_PAD - 1, jnp.int32)])
    ea_p = jnp.concatenate([edge_attr, jnp.zeros((pad, D_EDGE), jnp.float32)])

    W1 = W_eout[:D_NODE]
    W2 = W_eout[D_NODE:]
    Wn1 = W_node[:D_NODE]
    Wn2 = W_node[D_NODE:]
    c_attn = jnp.sum(b_eattn * w_attn)[None].astype(jnp.float32)

    n_eb = E_PAD // BE
    a, e2a, e2b, e2c, e2d = pl.pallas_call(
        _edge_kernel,
        grid=(n_eb,),
        in_specs=[
            pl.BlockSpec((BE, D_EDGE), lambda i: (i, i * 0)),
            pl.BlockSpec((D_EDGE, H_DIM), lambda i: (i * 0, i * 0)),
            pl.BlockSpec((1, H_DIM), lambda i: (i * 0, i * 0)),
            pl.BlockSpec((H_DIM, H_DIM), lambda i: (i * 0, i * 0)),
            pl.BlockSpec((1, H_DIM), lambda i: (i * 0, i * 0)),
            pl.BlockSpec((D_EDGE, H_DIM), lambda i: (i * 0, i * 0)),
            pl.BlockSpec((1, H_DIM), lambda i: (i * 0, i * 0)),
            pl.BlockSpec((1,), lambda i: (i * 0,)),
        ],
        out_specs=[
            pl.BlockSpec((BE,), lambda i: (i,)),
            pl.BlockSpec((BE, HH), lambda i: (i, i * 0)),
            pl.BlockSpec((BE, HH), lambda i: (i, i * 0)),
            pl.BlockSpec((BE, HH), lambda i: (i, i * 0)),
            pl.BlockSpec((BE, HH), lambda i: (i, i * 0)),
        ],
        out_shape=[
            jax.ShapeDtypeStruct((E_PAD,), jnp.float32),
            jax.ShapeDtypeStruct((E_PAD, HH), jnp.float32),
            jax.ShapeDtypeStruct((E_PAD, HH), jnp.float32),
            jax.ShapeDtypeStruct((E_PAD, HH), jnp.float32),
            jax.ShapeDtypeStruct((E_PAD, HH), jnp.float32),
        ],
    )(ea_p, W_edge, b_edge[None, :], W2, b_eout[None, :],
      W_eattn, w_attn[None, :], c_attn)

    n_nb = N_NODES // BN
    y1_halves = []
    for h0 in (0, HH, 2 * HH, 3 * HH):
        y1_halves.append(pl.pallas_call(
            _matmul_bias_kernel,
            grid=(n_nb,),
            in_specs=[
                pl.BlockSpec((BN, D_NODE), lambda i: (i, i * 0)),
                pl.BlockSpec((D_NODE, HH), lambda i: (i * 0, i * 0)),
                pl.BlockSpec((1, HH), lambda i: (i * 0, i * 0)),
            ],
            out_specs=pl.BlockSpec((BN, HH), lambda i: (i, i * 0)),
            out_shape=jax.ShapeDtypeStruct((N_NODES, HH), jnp.float32),
        )(x, W1[:, h0:h0 + HH], jnp.zeros((1, HH), jnp.float32)))
    y1a, y1b, y1c, y1d = y1_halves

    mesh = plsc.VectorSubcoreMesh(core_axis_name="c", subcore_axis_name="s")

    bisect = functools.partial(
        pl.kernel, mesh=mesh,
        compiler_params=pltpu.CompilerParams(needs_layout_passes=False,
                                             use_tc_tiling_on_sc=False),
        out_type=jax.ShapeDtypeStruct((NSEG_R, 16), jnp.float32),
        scratch_types=[
            pltpu.VMEM((BIS_TILE_R, 16), jnp.float32),
            pltpu.VMEM((BIS_TILE_R, 16), jnp.int32),
            pltpu.VMEM((NSEG_R, 16), jnp.float32),
            pltpu.VMEM((NSEG_R, 16), jnp.float32),
            pltpu.VMEM((NSEG_R, 16), jnp.float32),
            pltpu.VMEM((NSEG_R, 16), jnp.float32),
            pltpu.VMEM((NSEG_R, 16), jnp.float32),
            pltpu.VMEM((NSEG_R,), jnp.int32),
            pltpu.VMEM((NSEG_R // 16, 16), jnp.float32),
            pltpu.VMEM((1, 16), jnp.float32),
            pltpu.VMEM((16, 16), jnp.float32),
            pltpu.VMEM_SHARED((NSEG_R, 16), jnp.float32),
            pltpu.VMEM_SHARED((16, 16), jnp.float32),
        ],
    )(_bisect_kernel)
    tau = bisect(a.reshape(E_PAD // 16, 16), dst_p.reshape(E_PAD // 16, 16))

    message = functools.partial(
        pl.kernel, mesh=mesh,
        compiler_params=pltpu.CompilerParams(needs_layout_passes=False,
                                             use_tc_tiling_on_sc=False),
        out_type=jax.ShapeDtypeStruct((2, 4, N_PAD, HH), jnp.float32),
        scratch_types=[
            pltpu.VMEM((NSEG_R, 16), jnp.float32),
            pltpu.VMEM((MSG_CHUNKS * 128,), jnp.float32),
            pltpu.VMEM((MSG_CHUNKS * 128,), jnp.int32),
            pltpu.VMEM((MSG_CHUNKS * 128,), jnp.int32),
            pltpu.VMEM((MSG_CHUNKS * 128 + 144,), jnp.float32),
            pltpu.VMEM((MSG_CHUNKS * 128 + 144,), jnp.int32),
            pltpu.VMEM((MSG_CHUNKS * 128 + 144,), jnp.int32),
            pltpu.VMEM((MSG_CHUNKS * 128 + 144,), jnp.int32),
            pltpu.VMEM((128,), jnp.int32),
            pltpu.VMEM((128, HH), jnp.float32),
            pltpu.VMEM((128, HH), jnp.float32),
            pltpu.VMEM((128, HH), jnp.float32),
            pltpu.VMEM_SHARED((N_PAD, HH), jnp.float32),
            pltpu.SemaphoreType.DMA,
            pltpu.SemaphoreType.DMA,
        ],
    )(_message_kernel)
    hn = message(y1a, y1b, y1c, y1d, e2a, e2b, e2c, e2d, a,
                 src_p, dst_p, tau)

    def _hn_spec(cc, qq):
        return pl.BlockSpec(
            (1, 1, BN, HH),
            lambda i, _c=cc, _q=qq: (_c + i * 0, _q + i * 0, i, i * 0))

    h = pl.pallas_call(
        _final_kernel,
        grid=(n_nb,),
        in_specs=(
            [pl.BlockSpec((BN, D_NODE), lambda i: (i, i * 0))]
            + [_hn_spec(cc, qq) for cc in (0, 1) for qq in range(4)]
            + [pl.BlockSpec((HH, H_DIM), lambda i: (i * 0, i * 0))
               for _ in range(4)]
            + [pl.BlockSpec((D_NODE, H_DIM), lambda i: (i * 0, i * 0)),
               pl.BlockSpec((1, H_DIM), lambda i: (i * 0, i * 0))]
        ),
        out_specs=pl.BlockSpec((BN, H_DIM), lambda i: (i, i * 0)),
        out_shape=jax.ShapeDtypeStruct((N_NODES, H_DIM), jnp.float32),
    )(x, hn, hn, hn, hn, hn, hn, hn, hn,
      Wn2[:HH], Wn2[HH:2 * HH], Wn2[2 * HH:3 * HH], Wn2[3 * HH:],
      Wn1, b_node[None, :])
    return h


# speedup vs baseline: 1.5019x; 1.5019x over previous
import functools

import jax
import jax.numpy as jnp
from jax import lax
from jax.experimental import pallas as pl
from jax.experimental.pallas import tpu as pltpu
from jax.experimental.pallas import tpu_sc as plsc

N_NODES = 10000
E_EDGES = 320000
D_NODE = 128
D_EDGE = 16
H_DIM = 128
HH = H_DIM // 4

N_PAD = 10240
E_PAD = 327680
BE = 4096
BN = 2000
BISECT_ITERS = 24

NSEG_R = N_PAD // 16
BIS_TILE_R = E_PAD // 16 // 16
MSG_CHUNKS = E_PAD // 32 // 128


def _i32(v):
    return jnp.asarray(v, jnp.int32)


def _edge_kernel(ea_ref, We_ref, be_ref, W2_ref, b2_ref, Wa_ref, wa_ref, ca_ref,
                 a_ref, e2a_ref, e2b_ref, e2c_ref, e2d_ref):
    ea = ea_ref[...]
    eo = jnp.maximum(jnp.dot(ea, We_ref[...],
                             preferred_element_type=jnp.float32) + be_ref[...], 0.0)
    e2 = jnp.dot(eo, W2_ref[...],
                 preferred_element_type=jnp.float32) + b2_ref[...]
    e2a_ref[...] = e2[:, :HH]
    e2b_ref[...] = e2[:, HH:2 * HH]
    e2c_ref[...] = e2[:, 2 * HH:3 * HH]
    e2d_ref[...] = e2[:, 3 * HH:]
    v = jnp.sum(Wa_ref[...] * wa_ref[...], axis=1)
    aa = jnp.sum(ea * v[None, :], axis=1) + ca_ref[0]
    a_ref[...] = jnp.where(aa > 0, aa, 0.01 * aa)


def _matmul_bias_kernel(x_ref, W_ref, b_ref, o_ref):
    o_ref[...] = jnp.dot(x_ref[...], W_ref[...],
                         preferred_element_type=jnp.float32) + b_ref[...]


def _final_kernel(x_ref, hn0_ref, hn1_ref, hn2_ref, hn3_ref,
                  hn4_ref, hn5_ref, hn6_ref, hn7_ref,
                  W2q0_ref, W2q1_ref, W2q2_ref, W2q3_ref,
                  W1_ref, b_ref, o_ref):
    acc = jnp.dot(x_ref[...], W1_ref[...], preferred_element_type=jnp.float32)
    acc += jnp.dot(hn0_ref[0, 0] + hn4_ref[0, 0], W2q0_ref[...],
                   preferred_element_type=jnp.float32)
    acc += jnp.dot(hn1_ref[0, 0] + hn5_ref[0, 0], W2q1_ref[...],
                   preferred_element_type=jnp.float32)
    acc += jnp.dot(hn2_ref[0, 0] + hn6_ref[0, 0], W2q2_ref[...],
                   preferred_element_type=jnp.float32)
    acc += jnp.dot(hn3_ref[0, 0] + hn7_ref[0, 0], W2q3_ref[...],
                   preferred_element_type=jnp.float32)
    o_ref[...] = jnp.maximum(acc + b_ref[...], 0.0)


def _bisect_kernel(a16, d16, tau_hbm,
                   a_loc, d_loc, mid_loc, s_loc, lo_loc, hi_loc, zero_loc,
                   idx_loc, tau_loc, mm_loc, mmall_loc, s_sh, mm_sh):
    c = lax.axis_index("c")
    w = lax.axis_index("s")
    iota = jnp.arange(16, dtype=jnp.int32)

    pltpu.sync_copy(a16.at[pl.ds(w * BIS_TILE_R, BIS_TILE_R)], a_loc)
    pltpu.sync_copy(d16.at[pl.ds(w * BIS_TILE_R, BIS_TILE_R)], d_loc)

    def init_body(g, _):
        zero_loc[g] = jnp.zeros((16,), jnp.float32)
        return 0
    lax.fori_loop(0, NSEG_R, init_body, 0)

    def idx_body(g, vec):
        idx_loc[pl.ds(g * 16, 16)] = vec
        return vec + 16
    lax.fori_loop(0, NSEG_R // 16, idx_body, iota)

    def mm_body(g, carry):
        mn, mx = carry
        av = a_loc[g]
        return jnp.minimum(mn, av), jnp.maximum(mx, av)
    mn, mx = lax.fori_loop(0, BIS_TILE_R, mm_body,
                           (jnp.full((16,), jnp.inf, jnp.float32),
                            jnp.full((16,), -jnp.inf, jnp.float32)))
    gmn = jnp.min(mn)
    gmx = jnp.max(mx)
    mm_loc[0] = jnp.where(iota == 0, gmn, -gmx)
    pltpu.sync_copy(mm_loc, mm_sh.at[pl.ds(w, 1)])
    plsc.subcore_barrier()
    pltpu.sync_copy(mm_sh, mmall_loc)

    def mm_red(t, acc):
        return jnp.minimum(acc, mmall_loc[t])
    acc = lax.fori_loop(0, 16, mm_red, jnp.full((16,), jnp.inf, jnp.float32))
    inf = jnp.float32(jnp.inf)
    gmin = jnp.min(jnp.where(iota == 0, acc, inf))
    gmax = -jnp.min(jnp.where(iota == 1, acc, inf))

    def lohi_body(g, _):
        lo_loc[g] = jnp.full((16,), gmin - 1.0, jnp.float32)
        hi_loc[g] = jnp.full((16,), gmax, jnp.float32)
        mid_loc[g] = jnp.full((16,), 0.5 * (gmin - 1.0 + gmax), jnp.float32)
        s_loc[g] = jnp.zeros((16,), jnp.float32)
        return 0
    lax.fori_loop(0, NSEG_R, lohi_body, 0)

    def iter_body(_, carry):
        def edge_body(g, _c):
            idxv = d_loc[g]
            av = a_loc[g]
            row = jax.lax.shift_right_logical(idxv, _i32(4))
            lane = jnp.bitwise_and(idxv, _i32(15))
            mv = plsc.load_gather(mid_loc, [row, lane])
            contrib = jnp.maximum(av - mv, 0.0)
            plsc.addupdate_scatter(s_loc, [row, lane], contrib)
            return 0
        lax.fori_loop(0, BIS_TILE_R, edge_body, 0)

        plsc.subcore_barrier()

        @pl.when(w == 0)
        def _zero():
            pltpu.sync_copy(zero_loc, s_sh)
        plsc.subcore_barrier()
        pltpu.sync_copy(s_loc, s_sh.at[idx_loc], add=True)
        plsc.subcore_barrier()
        pltpu.sync_copy(s_sh, s_loc)

        def upd_body(g, _c):
            ge = s_loc[g] >= 1.0
            midv = mid_loc[g]
            lo = jnp.where(ge, midv, lo_loc[g])
            hi = jnp.where(ge, hi_loc[g], midv)
            lo_loc[g] = lo
            hi_loc[g] = hi
            mid_loc[g] = 0.5 * (lo + hi)
            s_loc[g] = jnp.zeros((16,), jnp.float32)
            return 0
        lax.fori_loop(0, NSEG_R, upd_body, 0)
        return 0

    lax.fori_loop(0, BISECT_ITERS, iter_body, 0)

    def tau_body(j, _c):
        g = w * (NSEG_R // 16) + j
        tau_loc[j] = 0.5 * (lo_loc[g] + hi_loc[g])
        return 0
    lax.fori_loop(0, NSEG_R // 16, tau_body, 0)

    @pl.when(c == 0)
    def _write():
        pltpu.sync_copy(tau_loc, tau_hbm.at[pl.ds(w * (NSEG_R // 16),
                                                  NSEG_R // 16)])


def _message_kernel(y1a_hbm, y1b_hbm, y1c_hbm, y1d_hbm,
                    e2a_hbm, e2b_hbm, e2c_hbm, e2d_hbm, a1_hbm, src1_hbm,
                    dst1_hbm, tau_hbm, hn_hbm,
                    tau_loc, a_loc, src_loc, dst_loc, alpha_c, src_c, dst_c,
                    eid_c, dst_chunk, g_buf, e_buf, zero_big, hn_sh,
                    sem, sem2):
    c = lax.axis_index("c")
    s_ = lax.axis_index("s")
    wid = c * 16 + s_
    e_base = wid * (MSG_CHUNKS * 128)
    iota = jnp.arange(16, dtype=jnp.int32)

    pltpu.sync_copy(tau_hbm, tau_loc)
    pltpu.sync_copy(a1_hbm.at[pl.ds(e_base, MSG_CHUNKS * 128)], a_loc)
    pltpu.sync_copy(src1_hbm.at[pl.ds(e_base, MSG_CHUNKS * 128)], src_loc)
    pltpu.sync_copy(dst1_hbm.at[pl.ds(e_base, MSG_CHUNKS * 128)], dst_loc)

    def zb(i, _c):
        def zq(q, _cc):
            zero_big[i, pl.ds(q * 16, 16)] = jnp.zeros((16,), jnp.float32)
            return 0
        lax.fori_loop(0, HH // 16, zq, 0)
        return 0
    lax.fori_loop(0, 128, zb, 0)

    def comp_body(g, carry):
        pos, ev = carry
        dv = dst_loc[pl.ds(g * 16, 16)]
        av = a_loc[pl.ds(g * 16, 16)]
        sv = src_loc[pl.ds(g * 16, 16)]
        row = jax.lax.shift_right_logical(dv, _i32(4))
        lane = jnp.bitwise_and(dv, _i32(15))
        tv = plsc.load_gather(tau_loc, [row, lane])
        alpha = jnp.maximum(av - tv, 0.0)
        mask = jnp.logical_and(alpha > 0.0, ev < E_EDGES)
        plsc.store_compressed(alpha_c.at[pl.ds(pos, 16)], x=alpha, mask=mask)
        plsc.store_compressed(src_c.at[pl.ds(pos, 16)], x=sv, mask=mask)
        plsc.store_compressed(dst_c.at[pl.ds(pos, 16)], x=dv, mask=mask)
        plsc.store_compressed(eid_c.at[pl.ds(pos, 16)], x=ev, mask=mask)
        npos = pos + jnp.max(plsc.all_reduce_population_count(mask))
        return npos, ev + 16
    cnt, _ = lax.fori_loop(
        0, MSG_CHUNKS * 8, comp_body,
        (_i32(0), jnp.full((16,), e_base, jnp.int32) + iota))

    def pad_body(j, _c):
        at = pl.ds(cnt + j * 16, 16)
        alpha_c[at] = jnp.zeros((16,), jnp.float32)
        src_c[at] = jnp.zeros((16,), jnp.int32)
        dst_c[at] = jnp.full((16,), N_PAD - 1, jnp.int32)
        eid_c[at] = jnp.zeros((16,), jnp.int32)
        return 0
    lax.fori_loop(0, 8, pad_body, 0)

    for ha, (y1h, e2h) in enumerate(((y1a_hbm, e2a_hbm), (y1b_hbm, e2b_hbm),
                                     (y1c_hbm, e2c_hbm), (y1d_hbm, e2d_hbm))):
        def zs(j, _c):
            pltpu.sync_copy(zero_big,
                            hn_sh.at[pl.ds(s_ * 640 + j * 128, 128)])
            return 0
        lax.fori_loop(0, 5, zs, 0)
        plsc.subcore_barrier()

        def chunk_body(ch, _c):
            @pl.when(ch * 128 < cnt)
            def _do():
                def cp(l, _cc):
                    dst_chunk[pl.ds(l * 16, 16)] = (
                        dst_c[pl.ds(ch * 128 + l * 16, 16)])
                    return 0
                lax.fori_loop(0, 8, cp, 0)

                cp1 = pltpu.async_copy(
                    y1h.at[src_c.at[pl.ds(ch * 128, 128)]], g_buf, sem)
                cp2 = pltpu.async_copy(
                    e2h.at[eid_c.at[pl.ds(ch * 128, 128)]], e_buf, sem2)
                cp1.wait()
                cp2.wait()

                def rowb(r, rfull):
                    ar = plsc.load_gather(alpha_c.at[pl.ds(ch * 128, 128)],
                                          [rfull])

                    def qb(q, _ccc):
                        mv = jnp.maximum(g_buf[r, pl.ds(q * 16, 16)]
                                         + e_buf[r, pl.ds(q * 16, 16)],
                                         0.0) * ar
                        g_buf[r, pl.ds(q * 16, 16)] = mv
                        return 0
                    lax.fori_loop(0, HH // 16, qb, 0)
                    return rfull + 1
                lax.fori_loop(0, 128, rowb, jnp.zeros((16,), jnp.int32))

                pltpu.sync_copy(g_buf, hn_sh.at[dst_chunk], add=True)
            return 0
        lax.fori_loop(0, MSG_CHUNKS, chunk_body, 0)

        plsc.subcore_barrier()
        pltpu.sync_copy(hn_sh.at[pl.ds(s_ * 640, 640)],
                        hn_hbm.at[c, ha, pl.ds(s_ * 640, 640)])
        plsc.subcore_barrier()


def kernel(x, edge_index, edge_attr, W_edge, b_edge, W_eattn, b_eattn, w_attn,
           W_eout, b_eout, W_node, b_node):
    edge_index = edge_index.astype(jnp.int32)
    with jax.enable_x64(False):
        return _kernel_impl(x, edge_index, edge_attr, W_edge, b_edge, W_eattn,
                            b_eattn, w_attn, W_eout, b_eout, W_node, b_node)


def _kernel_impl(x, edge_index, edge_attr, W_edge, b_edge, W_eattn, b_eattn,
                 w_attn, W_eout, b_eout, W_node, b_node):
    x = x.astype(jnp.float32)
    src = edge_index[0]
    dst = edge_index[1]
    edge_attr = edge_attr.astype(jnp.float32)

    pad = E_PAD - E_EDGES
    src_p = jnp.concatenate([src, jnp.zeros((pad,), jnp.int32)])
    dst_p = jnp.concatenate([dst, jnp.full((pad,), N_PAD - 1, jnp.int32)])
    ea_p = jnp.concatenate([edge_attr, jnp.zeros((pad, D_EDGE), jnp.float32)])

    W1 = W_eout[:D_NODE]
    W2 = W_eout[D_NODE:]
    Wn1 = W_node[:D_NODE]
    Wn2 = W_node[D_NODE:]
    c_attn = jnp.sum(b_eattn * w_attn)[None].astype(jnp.float32)

    n_eb = E_PAD // BE
    a, e2a, e2b, e2c, e2d = pl.pallas_call(
        _edge_kernel,
        grid=(n_eb,),
        in_specs=[
            pl.BlockSpec((BE, D_EDGE), lambda i: (i, i * 0)),
            pl.BlockSpec((D_EDGE, H_DIM), lambda i: (i * 0, i * 0)),
            pl.BlockSpec((1, H_DIM), lambda i: (i * 0, i * 0)),
            pl.BlockSpec((H_DIM, H_DIM), lambda i: (i * 0, i * 0)),
            pl.BlockSpec((1, H_DIM), lambda i: (i * 0, i * 0)),
            pl.BlockSpec((D_EDGE, H_DIM), lambda i: (i * 0, i * 0)),
            pl.BlockSpec((1, H_DIM), lambda i: (i * 0, i * 0)),
            pl.BlockSpec((1,), lambda i: (i * 0,)),
        ],
        out_specs=[
            pl.BlockSpec((BE,), lambda i: (i,)),
            pl.BlockSpec((BE, HH), lambda i: (i, i * 0)),
            pl.BlockSpec((BE, HH), lambda i: (i, i * 0)),
            pl.BlockSpec((BE, HH), lambda i: (i, i * 0)),
            pl.BlockSpec((BE, HH), lambda i: (i, i * 0)),
        ],
        out_shape=[
            jax.ShapeDtypeStruct((E_PAD,), jnp.float32),
            jax.ShapeDtypeStruct((E_PAD, HH), jnp.float32),
            jax.ShapeDtypeStruct((E_PAD, HH), jnp.float32),
            jax.ShapeDtypeStruct((E_PAD, HH), jnp.float32),
            jax.ShapeDtypeStruct((E_PAD, HH), jnp.float32),
        ],
    )(ea_p, W_edge, b_edge[None, :], W2, b_eout[None, :],
      W_eattn, w_attn[None, :], c_attn)

    n_nb = N_NODES // BN
    y1_halves = []
    for h0 in (0, HH, 2 * HH, 3 * HH):
        y1_halves.append(pl.pallas_call(
            _matmul_bias_kernel,
            grid=(n_nb,),
            in_specs=[
                pl.BlockSpec((BN, D_NODE), lambda i: (i, i * 0)),
                pl.BlockSpec((D_NODE, HH), lambda i: (i * 0, i * 0)),
                pl.BlockSpec((1, HH), lambda i: (i * 0, i * 0)),
            ],
            out_specs=pl.BlockSpec((BN, HH), lambda i: (i, i * 0)),
            out_shape=jax.ShapeDtypeStruct((N_NODES, HH), jnp.float32),
        )(x, W1[:, h0:h0 + HH], jnp.zeros((1, HH), jnp.float32)))
    y1a, y1b, y1c, y1d = y1_halves

    mesh = plsc.VectorSubcoreMesh(core_axis_name="c", subcore_axis_name="s")

    bisect = functools.partial(
        pl.kernel, mesh=mesh,
        compiler_params=pltpu.CompilerParams(needs_layout_passes=False,
                                             use_tc_tiling_on_sc=False),
        out_type=jax.ShapeDtypeStruct((NSEG_R, 16), jnp.float32),
        scratch_types=[
            pltpu.VMEM((BIS_TILE_R, 16), jnp.float32),
            pltpu.VMEM((BIS_TILE_R, 16), jnp.int32),
            pltpu.VMEM((NSEG_R, 16), jnp.float32),
            pltpu.VMEM((NSEG_R, 16), jnp.float32),
            pltpu.VMEM((NSEG_R, 16), jnp.float32),
            pltpu.VMEM((NSEG_R, 16), jnp.float32),
            pltpu.VMEM((NSEG_R, 16), jnp.float32),
            pltpu.VMEM((NSEG_R,), jnp.int32),
            pltpu.VMEM((NSEG_R // 16, 16), jnp.float32),
            pltpu.VMEM((1, 16), jnp.float32),
            pltpu.VMEM((16, 16), jnp.float32),
            pltpu.VMEM_SHARED((NSEG_R, 16), jnp.float32),
            pltpu.VMEM_SHARED((16, 16), jnp.float32),
        ],
    )(_bisect_kernel)
    tau = bisect(a.reshape(E_PAD // 16, 16), dst_p.reshape(E_PAD // 16, 16))

    message = functools.partial(
        pl.kernel, mesh=mesh,
        compiler_params=pltpu.CompilerParams(needs_layout_passes=False,
                                             use_tc_tiling_on_sc=False),
        out_type=jax.ShapeDtypeStruct((2, 4, N_PAD, HH), jnp.float32),
        scratch_types=[
            pltpu.VMEM((NSEG_R, 16), jnp.float32),
            pltpu.VMEM((MSG_CHUNKS * 128,), jnp.float32),
            pltpu.VMEM((MSG_CHUNKS * 128,), jnp.int32),
            pltpu.VMEM((MSG_CHUNKS * 128,), jnp.int32),
            pltpu.VMEM((MSG_CHUNKS * 128 + 144,), jnp.float32),
            pltpu.VMEM((MSG_CHUNKS * 128 + 144,), jnp.int32),
            pltpu.VMEM((MSG_CHUNKS * 128 + 144,), jnp.int32),
            pltpu.VMEM((MSG_CHUNKS * 128 + 144,), jnp.int32),
            pltpu.VMEM((128,), jnp.int32),
            pltpu.VMEM((128, HH), jnp.float32),
            pltpu.VMEM((128, HH), jnp.float32),
            pltpu.VMEM((128, HH), jnp.float32),
            pltpu.VMEM_SHARED((N_PAD, HH), jnp.float32),
            pltpu.SemaphoreType.DMA,
            pltpu.SemaphoreType.DMA,
        ],
    )(_message_kernel)
    hn = message(y1a, y1b, y1c, y1d, e2a, e2b, e2c, e2d, a,
                 src_p, dst_p, tau)

    def _hn_spec(cc, qq):
        return pl.BlockSpec(
            (1, 1, BN, HH),
            lambda i, _c=cc, _q=qq: (_c + i * 0, _q + i * 0, i, i * 0))

    h = pl.pallas_call(
        _final_kernel,
        grid=(n_nb,),
        in_specs=(
            [pl.BlockSpec((BN, D_NODE), lambda i: (i, i * 0))]
            + [_hn_spec(cc, qq) for cc in (0, 1) for qq in range(4)]
            + [pl.BlockSpec((HH, H_DIM), lambda i: (i * 0, i * 0))
               for _ in range(4)]
            + [pl.BlockSpec((D_NODE, H_DIM), lambda i: (i * 0, i * 0)),
               pl.BlockSpec((1, H_DIM), lambda i: (i * 0, i * 0))]
        ),
        out_specs=pl.BlockSpec((BN, H_DIM), lambda i: (i, i * 0)),
        out_shape=jax.ShapeDtypeStruct((N_NODES, H_DIM), jnp.float32),
    )(x, hn, hn, hn, hn, hn, hn, hn, hn,
      Wn2[:HH], Wn2[HH:2 * HH], Wn2[2 * HH:3 * HH], Wn2[3 * HH:],
      Wn1, b_node[None, :])
    return h


# speedup vs baseline: 1.6271x; 1.0834x over previous
import functools

import jax
import jax.numpy as jnp
from jax import lax
from jax.experimental import pallas as pl
from jax.experimental.pallas import tpu as pltpu
from jax.experimental.pallas import tpu_sc as plsc

N_NODES = 10000
E_EDGES = 320000
D_NODE = 128
D_EDGE = 16
H_DIM = 128
HH = H_DIM // 4

N_PAD = 10240
E_PAD = 327680
BE = 4096
BN = 2000
BISECT_ITERS = 18

NSEG_R = N_PAD // 16
BIS_TILE_R = E_PAD // 16 // 16
MSG_CHUNKS = E_PAD // 32 // 128


def _i32(v):
    return jnp.asarray(v, jnp.int32)


def _edge_kernel(ea_ref, We_ref, be_ref, W2_ref, b2_ref, Wa_ref, wa_ref, ca_ref,
                 a_ref, e2a_ref, e2b_ref, e2c_ref, e2d_ref):
    ea = ea_ref[...]
    eo = jnp.maximum(jnp.dot(ea, We_ref[...],
                             preferred_element_type=jnp.float32) + be_ref[...], 0.0)
    e2 = jnp.dot(eo, W2_ref[...],
                 preferred_element_type=jnp.float32) + b2_ref[...]
    e2a_ref[...] = e2[:, :HH]
    e2b_ref[...] = e2[:, HH:2 * HH]
    e2c_ref[...] = e2[:, 2 * HH:3 * HH]
    e2d_ref[...] = e2[:, 3 * HH:]
    v = jnp.sum(Wa_ref[...] * wa_ref[...], axis=1)
    aa = jnp.sum(ea * v[None, :], axis=1) + ca_ref[0]
    a_ref[...] = jnp.where(aa > 0, aa, 0.01 * aa)


def _matmul_bias_kernel(x_ref, W_ref, b_ref, o_ref):
    o_ref[...] = jnp.dot(x_ref[...], W_ref[...],
                         preferred_element_type=jnp.float32) + b_ref[...]


def _final_kernel(x_ref, hn0_ref, hn1_ref, hn2_ref, hn3_ref,
                  hn4_ref, hn5_ref, hn6_ref, hn7_ref,
                  W2q0_ref, W2q1_ref, W2q2_ref, W2q3_ref,
                  W1_ref, b_ref, o_ref):
    acc = jnp.dot(x_ref[...], W1_ref[...], preferred_element_type=jnp.float32)
    acc += jnp.dot(hn0_ref[0, 0] + hn4_ref[0, 0], W2q0_ref[...],
                   preferred_element_type=jnp.float32)
    acc += jnp.dot(hn1_ref[0, 0] + hn5_ref[0, 0], W2q1_ref[...],
                   preferred_element_type=jnp.float32)
    acc += jnp.dot(hn2_ref[0, 0] + hn6_ref[0, 0], W2q2_ref[...],
                   preferred_element_type=jnp.float32)
    acc += jnp.dot(hn3_ref[0, 0] + hn7_ref[0, 0], W2q3_ref[...],
                   preferred_element_type=jnp.float32)
    o_ref[...] = jnp.maximum(acc + b_ref[...], 0.0)


def _bisect_kernel(a16, d16, tau_hbm,
                   a_loc, d_loc, l_loc, mid_loc, s_loc, lo_loc, hi_loc,
                   zero_loc, idx_loc, tau_loc, mm_loc, mmall_loc,
                   s_sh0, s_sh1, mm_sh):
    c = lax.axis_index("c")
    w = lax.axis_index("s")
    iota = jnp.arange(16, dtype=jnp.int32)

    pltpu.sync_copy(a16.at[pl.ds(w * BIS_TILE_R, BIS_TILE_R)], a_loc)
    pltpu.sync_copy(d16.at[pl.ds(w * BIS_TILE_R, BIS_TILE_R)], d_loc)

    def init_body(g, _):
        zero_loc[g] = jnp.zeros((16,), jnp.float32)
        return 0
    lax.fori_loop(0, NSEG_R, init_body, 0)

    def split_body(g, _):
        dv = d_loc[g]
        d_loc[g] = jax.lax.shift_right_logical(dv, _i32(4))
        l_loc[g] = jnp.bitwise_and(dv, _i32(15))
        return 0
    lax.fori_loop(0, BIS_TILE_R, split_body, 0)

    def idx_body(g, vec):
        idx_loc[pl.ds(g * 16, 16)] = vec
        return vec + 16
    lax.fori_loop(0, NSEG_R // 16, idx_body, iota)

    def mm_body(g, carry):
        mn, mx = carry
        av = a_loc[g]
        return jnp.minimum(mn, av), jnp.maximum(mx, av)
    mn, mx = lax.fori_loop(0, BIS_TILE_R, mm_body,
                           (jnp.full((16,), jnp.inf, jnp.float32),
                            jnp.full((16,), -jnp.inf, jnp.float32)))
    gmn = jnp.min(mn)
    gmx = jnp.max(mx)
    mm_loc[0] = jnp.where(iota == 0, gmn, -gmx)
    pltpu.sync_copy(mm_loc, mm_sh.at[pl.ds(w, 1)])
    plsc.subcore_barrier()
    pltpu.sync_copy(mm_sh, mmall_loc)

    def mm_red(t, acc):
        return jnp.minimum(acc, mmall_loc[t])
    acc = lax.fori_loop(0, 16, mm_red, jnp.full((16,), jnp.inf, jnp.float32))
    inf = jnp.float32(jnp.inf)
    gmin = jnp.min(jnp.where(iota == 0, acc, inf))
    gmax = -jnp.min(jnp.where(iota == 1, acc, inf))

    def lohi_body(g, _):
        lo_loc[g] = jnp.full((16,), gmin - 1.0, jnp.float32)
        hi_loc[g] = jnp.full((16,), gmax, jnp.float32)
        mid_loc[g] = jnp.full((16,), 0.5 * (gmin - 1.0 + gmax), jnp.float32)
        s_loc[g] = jnp.zeros((16,), jnp.float32)
        return 0
    lax.fori_loop(0, NSEG_R, lohi_body, 0)

    @pl.when(w == 0)
    def _z0():
        pltpu.sync_copy(zero_loc, s_sh0)

    def iter_body(it, carry):
        def edge_body(g, _c):
            row = d_loc[g]
            lane = l_loc[g]
            av = a_loc[g]
            mv = plsc.load_gather(mid_loc, [row, lane])
            contrib = jnp.maximum(av - mv, 0.0)
            plsc.addupdate_scatter(s_loc, [row, lane], contrib)
            return 0
        lax.fori_loop(0, BIS_TILE_R, edge_body, 0)

        even = jnp.bitwise_and(it, _i32(1)) == 0
        plsc.subcore_barrier()

        @pl.when(even)
        def _red0():
            pltpu.sync_copy(s_loc, s_sh0.at[idx_loc], add=True)

        @pl.when(jnp.logical_not(even))
        def _red1():
            pltpu.sync_copy(s_loc, s_sh1.at[idx_loc], add=True)
        plsc.subcore_barrier()

        @pl.when(even)
        def _rd0():
            pltpu.sync_copy(s_sh0, s_loc)

        @pl.when(jnp.logical_not(even))
        def _rd1():
            pltpu.sync_copy(s_sh1, s_loc)

        @pl.when(jnp.logical_and(w == 0, even))
        def _zn1():
            pltpu.sync_copy(zero_loc, s_sh1)

        @pl.when(jnp.logical_and(w == 0, jnp.logical_not(even)))
        def _zn0():
            pltpu.sync_copy(zero_loc, s_sh0)

        def upd_body(g, _c):
            ge = s_loc[g] >= 1.0
            midv = mid_loc[g]
            lo = jnp.where(ge, midv, lo_loc[g])
            hi = jnp.where(ge, hi_loc[g], midv)
            lo_loc[g] = lo
            hi_loc[g] = hi
            mid_loc[g] = 0.5 * (lo + hi)
            s_loc[g] = jnp.zeros((16,), jnp.float32)
            return 0
        lax.fori_loop(0, NSEG_R, upd_body, 0)
        return 0

    lax.fori_loop(0, BISECT_ITERS, iter_body, 0)

    def tau_body(j, _c):
        g = w * (NSEG_R // 16) + j
        tau_loc[j] = 0.5 * (lo_loc[g] + hi_loc[g])
        return 0
    lax.fori_loop(0, NSEG_R // 16, tau_body, 0)

    @pl.when(c == 0)
    def _write():
        pltpu.sync_copy(tau_loc, tau_hbm.at[pl.ds(w * (NSEG_R // 16),
                                                  NSEG_R // 16)])


def _message_kernel(y1a_hbm, y1b_hbm, y1c_hbm, y1d_hbm,
                    e2a_hbm, e2b_hbm, e2c_hbm, e2d_hbm, a1_hbm, src1_hbm,
                    dst1_hbm, tau_hbm, hn_hbm,
                    tau_loc, a_loc, src_loc, dst_loc, alpha_c, src_c, dst_c,
                    eid_c, dst_chunk, g_buf, e_buf, zero_big, hn_sh,
                    sem, sem2):
    c = lax.axis_index("c")
    s_ = lax.axis_index("s")
    wid = c * 16 + s_
    e_base = wid * (MSG_CHUNKS * 128)
    iota = jnp.arange(16, dtype=jnp.int32)

    pltpu.sync_copy(tau_hbm, tau_loc)
    pltpu.sync_copy(a1_hbm.at[pl.ds(e_base, MSG_CHUNKS * 128)], a_loc)
    pltpu.sync_copy(src1_hbm.at[pl.ds(e_base, MSG_CHUNKS * 128)], src_loc)
    pltpu.sync_copy(dst1_hbm.at[pl.ds(e_base, MSG_CHUNKS * 128)], dst_loc)

    def zb(i, _c):
        def zq(q, _cc):
            zero_big[i, pl.ds(q * 16, 16)] = jnp.zeros((16,), jnp.float32)
            return 0
        lax.fori_loop(0, HH // 16, zq, 0)
        return 0
    lax.fori_loop(0, 128, zb, 0)

    def comp_body(g, carry):
        pos, ev = carry
        dv = dst_loc[pl.ds(g * 16, 16)]
        av = a_loc[pl.ds(g * 16, 16)]
        sv = src_loc[pl.ds(g * 16, 16)]
        row = jax.lax.shift_right_logical(dv, _i32(4))
        lane = jnp.bitwise_and(dv, _i32(15))
        tv = plsc.load_gather(tau_loc, [row, lane])
        alpha = jnp.maximum(av - tv, 0.0)
        mask = jnp.logical_and(alpha > 0.0, ev < E_EDGES)
        plsc.store_compressed(alpha_c.at[pl.ds(pos, 16)], x=alpha, mask=mask)
        plsc.store_compressed(src_c.at[pl.ds(pos, 16)], x=sv, mask=mask)
        plsc.store_compressed(dst_c.at[pl.ds(pos, 16)], x=dv, mask=mask)
        plsc.store_compressed(eid_c.at[pl.ds(pos, 16)], x=ev, mask=mask)
        npos = pos + jnp.max(plsc.all_reduce_population_count(mask))
        return npos, ev + 16
    cnt, _ = lax.fori_loop(
        0, MSG_CHUNKS * 8, comp_body,
        (_i32(0), jnp.full((16,), e_base, jnp.int32) + iota))

    def pad_body(j, _c):
        at = pl.ds(cnt + j * 16, 16)
        alpha_c[at] = jnp.zeros((16,), jnp.float32)
        src_c[at] = jnp.zeros((16,), jnp.int32)
        dst_c[at] = jnp.full((16,), N_PAD - 1, jnp.int32)
        eid_c[at] = jnp.zeros((16,), jnp.int32)
        return 0
    lax.fori_loop(0, 8, pad_body, 0)

    for ha, (y1h, e2h) in enumerate(((y1a_hbm, e2a_hbm), (y1b_hbm, e2b_hbm),
                                     (y1c_hbm, e2c_hbm), (y1d_hbm, e2d_hbm))):
        def zs(j, _c):
            pltpu.sync_copy(zero_big,
                            hn_sh.at[pl.ds(s_ * 640 + j * 128, 128)])
            return 0
        lax.fori_loop(0, 5, zs, 0)
        plsc.subcore_barrier()

        def chunk_body(ch, _c):
            @pl.when(ch * 128 < cnt)
            def _do():
                def cp(l, _cc):
                    dst_chunk[pl.ds(l * 16, 16)] = (
                        dst_c[pl.ds(ch * 128 + l * 16, 16)])
                    return 0
                lax.fori_loop(0, 8, cp, 0)

                cp1 = pltpu.async_copy(
                    y1h.at[src_c.at[pl.ds(ch * 128, 128)]], g_buf, sem)
                cp2 = pltpu.async_copy(
                    e2h.at[eid_c.at[pl.ds(ch * 128, 128)]], e_buf, sem2)
                cp1.wait()
                cp2.wait()

                def rowb(r, rfull):
                    ar = plsc.load_gather(alpha_c.at[pl.ds(ch * 128, 128)],
                                          [rfull])

                    def qb(q, _ccc):
                        mv = jnp.maximum(g_buf[r, pl.ds(q * 16, 16)]
                                         + e_buf[r, pl.ds(q * 16, 16)],
                                         0.0) * ar
                        g_buf[r, pl.ds(q * 16, 16)] = mv
                        return 0
                    lax.fori_loop(0, HH // 16, qb, 0)
                    return rfull + 1
                lax.fori_loop(0, 128, rowb, jnp.zeros((16,), jnp.int32))

                pltpu.sync_copy(g_buf, hn_sh.at[dst_chunk], add=True)
            return 0
        lax.fori_loop(0, MSG_CHUNKS, chunk_body, 0)

        plsc.subcore_barrier()
        pltpu.sync_copy(hn_sh.at[pl.ds(s_ * 640, 640)],
                        hn_hbm.at[c, ha, pl.ds(s_ * 640, 640)])
        plsc.subcore_barrier()


def kernel(x, edge_index, edge_attr, W_edge, b_edge, W_eattn, b_eattn, w_attn,
           W_eout, b_eout, W_node, b_node):
    edge_index = edge_index.astype(jnp.int32)
    with jax.enable_x64(False):
        return _kernel_impl(x, edge_index, edge_attr, W_edge, b_edge, W_eattn,
                            b_eattn, w_attn, W_eout, b_eout, W_node, b_node)


def _kernel_impl(x, edge_index, edge_attr, W_edge, b_edge, W_eattn, b_eattn,
                 w_attn, W_eout, b_eout, W_node, b_node):
    x = x.astype(jnp.float32)
    src = edge_index[0]
    dst = edge_index[1]
    edge_attr = edge_attr.astype(jnp.float32)

    pad = E_PAD - E_EDGES
    src_p = jnp.concatenate([src, jnp.zeros((pad,), jnp.int32)])
    dst_p = jnp.concatenate([dst, jnp.full((pad,), N_PAD - 1, jnp.int32)])
    ea_p = jnp.concatenate([edge_attr, jnp.zeros((pad, D_EDGE), jnp.float32)])

    W1 = W_eout[:D_NODE]
    W2 = W_eout[D_NODE:]
    Wn1 = W_node[:D_NODE]
    Wn2 = W_node[D_NODE:]
    c_attn = jnp.sum(b_eattn * w_attn)[None].astype(jnp.float32)

    n_eb = E_PAD // BE
    a, e2a, e2b, e2c, e2d = pl.pallas_call(
        _edge_kernel,
        grid=(n_eb,),
        in_specs=[
            pl.BlockSpec((BE, D_EDGE), lambda i: (i, i * 0)),
            pl.BlockSpec((D_EDGE, H_DIM), lambda i: (i * 0, i * 0)),
            pl.BlockSpec((1, H_DIM), lambda i: (i * 0, i * 0)),
            pl.BlockSpec((H_DIM, H_DIM), lambda i: (i * 0, i * 0)),
            pl.BlockSpec((1, H_DIM), lambda i: (i * 0, i * 0)),
            pl.BlockSpec((D_EDGE, H_DIM), lambda i: (i * 0, i * 0)),
            pl.BlockSpec((1, H_DIM), lambda i: (i * 0, i * 0)),
            pl.BlockSpec((1,), lambda i: (i * 0,)),
        ],
        out_specs=[
            pl.BlockSpec((BE,), lambda i: (i,)),
            pl.BlockSpec((BE, HH), lambda i: (i, i * 0)),
            pl.BlockSpec((BE, HH), lambda i: (i, i * 0)),
            pl.BlockSpec((BE, HH), lambda i: (i, i * 0)),
            pl.BlockSpec((BE, HH), lambda i: (i, i * 0)),
        ],
        out_shape=[
            jax.ShapeDtypeStruct((E_PAD,), jnp.float32),
            jax.ShapeDtypeStruct((E_PAD, HH), jnp.float32),
            jax.ShapeDtypeStruct((E_PAD, HH), jnp.float32),
            jax.ShapeDtypeStruct((E_PAD, HH), jnp.float32),
            jax.ShapeDtypeStruct((E_PAD, HH), jnp.float32),
        ],
    )(ea_p, W_edge, b_edge[None, :], W2, b_eout[None, :],
      W_eattn, w_attn[None, :], c_attn)

    n_nb = N_NODES // BN
    y1_halves = []
    for h0 in (0, HH, 2 * HH, 3 * HH):
        y1_halves.append(pl.pallas_call(
            _matmul_bias_kernel,
            grid=(n_nb,),
            in_specs=[
                pl.BlockSpec((BN, D_NODE), lambda i: (i, i * 0)),
                pl.BlockSpec((D_NODE, HH), lambda i: (i * 0, i * 0)),
                pl.BlockSpec((1, HH), lambda i: (i * 0, i * 0)),
            ],
            out_specs=pl.BlockSpec((BN, HH), lambda i: (i, i * 0)),
            out_shape=jax.ShapeDtypeStruct((N_NODES, HH), jnp.float32),
        )(x, W1[:, h0:h0 + HH], jnp.zeros((1, HH), jnp.float32)))
    y1a, y1b, y1c, y1d = y1_halves

    mesh = plsc.VectorSubcoreMesh(core_axis_name="c", subcore_axis_name="s")

    bisect = functools.partial(
        pl.kernel, mesh=mesh,
        compiler_params=pltpu.CompilerParams(needs_layout_passes=False,
                                             use_tc_tiling_on_sc=False),
        out_type=jax.ShapeDtypeStruct((NSEG_R, 16), jnp.float32),
        scratch_types=[
            pltpu.VMEM((BIS_TILE_R, 16), jnp.float32),
            pltpu.VMEM((BIS_TILE_R, 16), jnp.int32),
            pltpu.VMEM((BIS_TILE_R, 16), jnp.int32),
            pltpu.VMEM((NSEG_R, 16), jnp.float32),
            pltpu.VMEM((NSEG_R, 16), jnp.float32),
            pltpu.VMEM((NSEG_R, 16), jnp.float32),
            pltpu.VMEM((NSEG_R, 16), jnp.float32),
            pltpu.VMEM((NSEG_R, 16), jnp.float32),
            pltpu.VMEM((NSEG_R,), jnp.int32),
            pltpu.VMEM((NSEG_R // 16, 16), jnp.float32),
            pltpu.VMEM((1, 16), jnp.float32),
            pltpu.VMEM((16, 16), jnp.float32),
            pltpu.VMEM_SHARED((NSEG_R, 16), jnp.float32),
            pltpu.VMEM_SHARED((NSEG_R, 16), jnp.float32),
            pltpu.VMEM_SHARED((16, 16), jnp.float32),
        ],
    )(_bisect_kernel)
    tau = bisect(a.reshape(E_PAD // 16, 16), dst_p.reshape(E_PAD // 16, 16))

    message = functools.partial(
        pl.kernel, mesh=mesh,
        compiler_params=pltpu.CompilerParams(needs_layout_passes=False,
                                             use_tc_tiling_on_sc=False),
        out_type=jax.ShapeDtypeStruct((2, 4, N_PAD, HH), jnp.float32),
        scratch_types=[
            pltpu.VMEM((NSEG_R, 16), jnp.float32),
            pltpu.VMEM((MSG_CHUNKS * 128,), jnp.float32),
            pltpu.VMEM((MSG_CHUNKS * 128,), jnp.int32),
            pltpu.VMEM((MSG_CHUNKS * 128,), jnp.int32),
            pltpu.VMEM((MSG_CHUNKS * 128 + 144,), jnp.float32),
            pltpu.VMEM((MSG_CHUNKS * 128 + 144,), jnp.int32),
            pltpu.VMEM((MSG_CHUNKS * 128 + 144,), jnp.int32),
            pltpu.VMEM((MSG_CHUNKS * 128 + 144,), jnp.int32),
            pltpu.VMEM((128,), jnp.int32),
            pltpu.VMEM((128, HH), jnp.float32),
            pltpu.VMEM((128, HH), jnp.float32),
            pltpu.VMEM((128, HH), jnp.float32),
            pltpu.VMEM_SHARED((N_PAD, HH), jnp.float32),
            pltpu.SemaphoreType.DMA,
            pltpu.SemaphoreType.DMA,
        ],
    )(_message_kernel)
    hn = message(y1a, y1b, y1c, y1d, e2a, e2b, e2c, e2d, a,
                 src_p, dst_p, tau)

    def _hn_spec(cc, qq):
        return pl.BlockSpec(
            (1, 1, BN, HH),
            lambda i, _c=cc, _q=qq: (_c + i * 0, _q + i * 0, i, i * 0))

    h = pl.pallas_call(
        _final_kernel,
        grid=(n_nb,),
        in_specs=(
            [pl.BlockSpec((BN, D_NODE), lambda i: (i, i * 0))]
            + [_hn_spec(cc, qq) for cc in (0, 1) for qq in range(4)]
            + [pl.BlockSpec((HH, H_DIM), lambda i: (i * 0, i * 0))
               for _ in range(4)]
            + [pl.BlockSpec((D_NODE, H_DIM), lambda i: (i * 0, i * 0)),
               pl.BlockSpec((1, H_DIM), lambda i: (i * 0, i * 0))]
        ),
        out_specs=pl.BlockSpec((BN, H_DIM), lambda i: (i, i * 0)),
        out_shape=jax.ShapeDtypeStruct((N_NODES, H_DIM), jnp.float32),
    )(x, hn, hn, hn, hn, hn, hn, hn, hn,
      Wn2[:HH], Wn2[HH:2 * HH], Wn2[2 * HH:3 * HH], Wn2[3 * HH:],
      Wn1, b_node[None, :])
    return h
